# Initial kernel scaffold; baseline (speedup 1.0000x reference)
import jax
import jax.numpy as jnp
from jax.experimental import pallas as pl


def kernel(x, edge_index, W1_l, b1, W1_r, gamma, beta, W2_l, b2, W2_r):
    raise NotImplementedError("write your pallas kernel here")


# speedup vs baseline: 9.3721x; 9.3721x over previous
import functools

import jax
import jax.numpy as jnp
from jax import lax
from jax.experimental import pallas as pl
from jax.experimental.pallas import tpu as pltpu
from jax.experimental.pallas import tpu_sc as plsc

_NC = 2
_NS = 16
_W = 80


@functools.lru_cache(maxsize=None)
def _edge_agg(n, e, d, with_cnt):
    nw = _NC * _NS
    per_w = e // nw
    assert e % nw == 0 and per_w % _W == 0
    nwin = per_w // _W
    assert nwin % 2 == 1
    per_t = (n // _NS) // 8 * 8
    last_t = n - (_NS - 1) * per_t
    assert per_t % 8 == 0 and last_t > 0

    mesh = plsc.VectorSubcoreMesh(core_axis_name="c", subcore_axis_name="s")

    out_type = [jax.ShapeDtypeStruct((_NC, n, d), jnp.float32)]
    if with_cnt:
        out_type += [jax.ShapeDtypeStruct((n,), jnp.float32)] * 2

    scratch = [
        pltpu.VMEM((_W,), jnp.int32),
        pltpu.VMEM((_W,), jnp.int32),
        pltpu.VMEM((_W,), jnp.int32),
        pltpu.VMEM((_W,), jnp.int32),
        pltpu.VMEM((_W, d), jnp.float32),
        pltpu.VMEM((_W, d), jnp.float32),
        pltpu.SemaphoreType.DMA,
        pltpu.SemaphoreType.DMA,
        pltpu.VMEM_SHARED((n, d), jnp.float32),
    ]
    if with_cnt:
        scratch.append(pltpu.VMEM_SHARED((n,), jnp.float32))
        scratch.append(pltpu.VMEM((_W,), jnp.float32))

    def body(y_hbm, src_hbm, dst_hbm, zeros_nd, *rest):
        if with_cnt:
            (zeros_n, agg_out, cnt0_out, cnt1_out,
             sb0, sb1, db0, db1, rb0, rb1, s0, s1,
             agg_sh, cnt_sh, ones_v) = rest
        else:
            (agg_out, sb0, sb1, db0, db1, rb0, rb1, s0, s1, agg_sh) = rest
        cid = lax.axis_index("c")
        sid = lax.axis_index("s")
        ebase = (sid * _NC + cid) * per_w
        row0 = sid * per_t

        @pl.when(sid < _NS - 1)
        def _():
            pltpu.sync_copy(zeros_nd.at[pl.ds(row0, per_t)],
                            agg_sh.at[pl.ds(row0, per_t)])

        @pl.when(sid == _NS - 1)
        def _():
            pltpu.sync_copy(zeros_nd.at[pl.ds((_NS - 1) * per_t, last_t)],
                            agg_sh.at[pl.ds((_NS - 1) * per_t, last_t)])

        if with_cnt:
            @pl.when(sid == 0)
            def _():
                pltpu.sync_copy(zeros_n, cnt_sh)
            for t in range(_W // 16):
                ones_v[pl.ds(t * 16, 16)] = jnp.ones((16,), jnp.float32)
        plsc.subcore_barrier()

        sbufs = (sb0, sb1)
        dbufs = (db0, db1)
        rbufs = (rb0, rb1)
        sems = (s0, s1)

        def load_and_start(win, b):
            off = ebase + win * _W
            pltpu.sync_copy(src_hbm.at[pl.ds(off, _W)], sbufs[b])
            pltpu.sync_copy(dst_hbm.at[pl.ds(off, _W)], dbufs[b])
            pltpu.make_async_copy(y_hbm.at[sbufs[b]], rbufs[b],
                                  sems[b]).start()

        def finish(b):
            pltpu.make_async_copy(y_hbm.at[sbufs[b]], rbufs[b],
                                  sems[b]).wait()
            pltpu.sync_copy(rbufs[b], agg_sh.at[dbufs[b]], add=True)
            if with_cnt:
                pltpu.sync_copy(ones_v, cnt_sh.at[dbufs[b]], add=True)

        load_and_start(0, 0)

        def loop_body(i, carry):
            j = 2 * i
            load_and_start(j + 1, 1)
            finish(0)
            load_and_start(j + 2, 0)
            finish(1)
            return carry

        lax.fori_loop(0, (nwin - 1) // 2, loop_body, 0)
        finish(0)

        plsc.subcore_barrier()

        @pl.when(sid < _NS - 1)
        def _():
            pltpu.sync_copy(agg_sh.at[pl.ds(row0, per_t)],
                            agg_out.at[cid, pl.ds(row0, per_t)])

        @pl.when(sid == _NS - 1)
        def _():
            pltpu.sync_copy(agg_sh.at[pl.ds((_NS - 1) * per_t, last_t)],
                            agg_out.at[cid, pl.ds((_NS - 1) * per_t, last_t)])

        if with_cnt:
            @pl.when((sid == 0) & (cid == 0))
            def _():
                pltpu.sync_copy(cnt_sh, cnt0_out)

            @pl.when((sid == 0) & (cid == 1))
            def _():
                pltpu.sync_copy(cnt_sh, cnt1_out)

    return pl.kernel(
        body, mesh=mesh, out_type=out_type, scratch_types=scratch,
        compiler_params=pltpu.CompilerParams(use_tc_tiling_on_sc=False))


def _pre(x, w_l, b, w_r):
    n = x.shape[0]
    hid = w_l.shape[1]

    def body(x_ref, wl_ref, b_ref, wr_ref, y_ref, r_ref):
        xv = x_ref[...]
        y_ref[...] = jnp.dot(xv, wl_ref[...],
                             preferred_element_type=jnp.float32)
        r_ref[...] = jnp.dot(xv, wr_ref[...],
                             preferred_element_type=jnp.float32) + b_ref[...]

    return pl.pallas_call(
        body,
        out_shape=[jax.ShapeDtypeStruct((n, hid), jnp.float32)] * 2,
    )(x, w_l, b, w_r)


def _mid(agg, c0, c1, r1, gamma, beta, w2_l, b2, w2_r):
    n, hid = r1.shape
    out_d = w2_l.shape[1]

    def body(agg_ref, c0_ref, c1_ref, r1_ref, g_ref, be_ref, wl_ref, b2_ref,
             wr_ref, y2_ref, r2_ref):
        s = agg_ref[0] + agg_ref[1]
        c = c0_ref[...] + c1_ref[...]
        h = s / jnp.maximum(c, 1.0) + r1_ref[...]
        m = jnp.mean(h, axis=0)
        v = jnp.mean((h - m) ** 2, axis=0)
        hn = (h - m) * lax.rsqrt(v + 1e-5) * g_ref[...] + be_ref[...]
        ha = jnp.maximum(hn, 0.0)
        y2_ref[...] = jnp.dot(ha, wl_ref[...],
                              preferred_element_type=jnp.float32)
        r2_ref[...] = jnp.dot(ha, wr_ref[...],
                              preferred_element_type=jnp.float32) + b2_ref[...]

    return pl.pallas_call(
        body,
        out_shape=[jax.ShapeDtypeStruct((n, out_d), jnp.float32)] * 2,
    )(agg, c0, c1, r1, gamma, beta, w2_l, b2, w2_r)


def _post(agg, c0, c1, r2):
    def body(agg_ref, c0_ref, c1_ref, r2_ref, out_ref):
        s = agg_ref[0] + agg_ref[1]
        c = c0_ref[...] + c1_ref[...]
        out_ref[...] = s / jnp.maximum(c, 1.0) + r2_ref[...]

    return pl.pallas_call(
        body,
        out_shape=jax.ShapeDtypeStruct(r2.shape, jnp.float32),
    )(agg, c0, c1, r2)


def kernel(x, edge_index, W1_l, b1, W1_r, gamma, beta, W2_l, b2, W2_r):
    n = x.shape[0]
    e = edge_index.shape[1]
    hid = W1_l.shape[1]
    out_d = W2_l.shape[1]
    src = edge_index[0]
    dst = edge_index[1]

    y1, r1 = _pre(x, W1_l, b1, W1_r)
    agg1, cnt0, cnt1 = _edge_agg(n, e, hid, True)(
        y1, src, dst,
        jnp.zeros((n, hid), jnp.float32), jnp.zeros((n,), jnp.float32))
    c0 = cnt0.reshape(n, 1)
    c1 = cnt1.reshape(n, 1)
    y2, r2 = _mid(agg1, c0, c1, r1, gamma, beta, W2_l, b2, W2_r)
    (agg2,) = _edge_agg(n, e, out_d, False)(
        y2, src, dst, jnp.zeros((n, out_d), jnp.float32))
    return _post(agg2, c0, c1, r2)


# speedup vs baseline: 14.9195x; 1.5919x over previous
import functools

import jax
import jax.numpy as jnp
from jax import lax
from jax.experimental import pallas as pl
from jax.experimental.pallas import tpu as pltpu
from jax.experimental.pallas import tpu_sc as plsc

_NC = 2
_NS = 16
_W = 80
_K = 5


@functools.lru_cache(maxsize=None)
def _edge_agg(n, e, d, with_cnt):
    nw = _NC * _NS
    per_w = e // nw
    gw = _K * _W
    assert e % nw == 0 and per_w % gw == 0
    ngrp = per_w // gw
    per_t = (n // _NS) // 8 * 8
    last_t = n - (_NS - 1) * per_t
    assert per_t % 8 == 0 and last_t > 0

    mesh = plsc.VectorSubcoreMesh(core_axis_name="c", subcore_axis_name="s")

    out_type = [jax.ShapeDtypeStruct((_NC, n, d), jnp.float32)]
    if with_cnt:
        out_type += [jax.ShapeDtypeStruct((n,), jnp.float32)] * 2

    scratch = [
        pltpu.VMEM((_K, 2, _W), jnp.int32),
        pltpu.VMEM((_K * _W, d), jnp.float32),
        pltpu.SemaphoreType.DMA,
        pltpu.SemaphoreType.DMA,
        pltpu.SemaphoreType.DMA,
        pltpu.VMEM_SHARED((n, d), jnp.float32),
    ]
    if with_cnt:
        scratch.append(pltpu.VMEM_SHARED((n,), jnp.float32))
        scratch.append(pltpu.VMEM((_W,), jnp.float32))

    def body(y_hbm, eidx_hbm, zeros_nd, *rest):
        if with_cnt:
            (zeros_n, agg_out, cnt0_out, cnt1_out,
             ibuf, rbuf, isem, gsem, ssem, agg_sh, cnt_sh, ones_v) = rest
        else:
            (agg_out, ibuf, rbuf, isem, gsem, ssem, agg_sh) = rest
        cid = lax.axis_index("c")
        sid = lax.axis_index("s")
        ebase = (sid * _NC + cid) * per_w
        row0 = sid * per_t

        @pl.when(sid < _NS - 1)
        def _():
            pltpu.sync_copy(zeros_nd.at[pl.ds(row0, per_t)],
                            agg_sh.at[pl.ds(row0, per_t)])

        @pl.when(sid == _NS - 1)
        def _():
            pltpu.sync_copy(zeros_nd.at[pl.ds((_NS - 1) * per_t, last_t)],
                            agg_sh.at[pl.ds((_NS - 1) * per_t, last_t)])

        if with_cnt:
            @pl.when(sid == 0)
            def _():
                pltpu.sync_copy(zeros_n, cnt_sh)
            for t in range(_W // 16):
                ones_v[pl.ds(t * 16, 16)] = jnp.ones((16,), jnp.float32)
        plsc.subcore_barrier()

        def do_group(g, carry):
            base = ebase + g * gw
            for k in range(_K):
                pltpu.async_copy(
                    eidx_hbm.at[:, pl.ds(base + k * _W, _W)],
                    ibuf.at[k], isem)
            for k in range(_K):
                pltpu.make_async_copy(
                    eidx_hbm.at[:, pl.ds(base + k * _W, _W)],
                    ibuf.at[k], isem).wait()
            for k in range(_K):
                pltpu.async_copy(y_hbm.at[ibuf.at[k, 0]],
                                 rbuf.at[pl.ds(k * _W, _W)], gsem)
            for k in range(_K):
                pltpu.make_async_copy(y_hbm.at[ibuf.at[k, 0]],
                                      rbuf.at[pl.ds(k * _W, _W)],
                                      gsem).wait()
            for k in range(_K):
                pltpu.async_copy(rbuf.at[pl.ds(k * _W, _W)],
                                 agg_sh.at[ibuf.at[k, 1]], ssem, add=True)
            if with_cnt:
                for k in range(_K):
                    pltpu.async_copy(ones_v, cnt_sh.at[ibuf.at[k, 1]],
                                     ssem, add=True)
            for k in range(_K):
                pltpu.make_async_copy(rbuf.at[pl.ds(k * _W, _W)],
                                      agg_sh.at[ibuf.at[k, 1]], ssem).wait()
            if with_cnt:
                for k in range(_K):
                    pltpu.make_async_copy(ones_v, cnt_sh.at[ibuf.at[k, 1]],
                                          ssem).wait()
            return carry

        lax.fori_loop(0, ngrp, do_group, 0)
        plsc.subcore_barrier()

        @pl.when(sid < _NS - 1)
        def _():
            pltpu.sync_copy(agg_sh.at[pl.ds(row0, per_t)],
                            agg_out.at[cid, pl.ds(row0, per_t)])

        @pl.when(sid == _NS - 1)
        def _():
            pltpu.sync_copy(agg_sh.at[pl.ds((_NS - 1) * per_t, last_t)],
                            agg_out.at[cid, pl.ds((_NS - 1) * per_t, last_t)])

        if with_cnt:
            @pl.when((sid == 0) & (cid == 0))
            def _():
                pltpu.sync_copy(cnt_sh, cnt0_out)

            @pl.when((sid == 0) & (cid == 1))
            def _():
                pltpu.sync_copy(cnt_sh, cnt1_out)

    return pl.kernel(
        body, mesh=mesh, out_type=out_type, scratch_types=scratch,
        compiler_params=pltpu.CompilerParams(use_tc_tiling_on_sc=False))


def _pre(x, w_l, b, w_r):
    n = x.shape[0]
    hid = w_l.shape[1]

    def body(x_ref, wl_ref, b_ref, wr_ref, y_ref, r_ref):
        xv = x_ref[...]
        y_ref[...] = jnp.dot(xv, wl_ref[...],
                             preferred_element_type=jnp.float32)
        r_ref[...] = jnp.dot(xv, wr_ref[...],
                             preferred_element_type=jnp.float32) + b_ref[...]

    return pl.pallas_call(
        body,
        out_shape=[jax.ShapeDtypeStruct((n, hid), jnp.float32)] * 2,
    )(x, w_l, b, w_r)


def _mid(agg, c0, c1, r1, gamma, beta, w2_l, b2, w2_r):
    n, hid = r1.shape
    out_d = w2_l.shape[1]

    def body(agg_ref, c0_ref, c1_ref, r1_ref, g_ref, be_ref, wl_ref, b2_ref,
             wr_ref, y2_ref, r2_ref):
        s = agg_ref[0] + agg_ref[1]
        c = c0_ref[...] + c1_ref[...]
        h = s / jnp.maximum(c, 1.0) + r1_ref[...]
        m = jnp.mean(h, axis=0)
        v = jnp.mean((h - m) ** 2, axis=0)
        hn = (h - m) * lax.rsqrt(v + 1e-5) * g_ref[...] + be_ref[...]
        ha = jnp.maximum(hn, 0.0)
        y2_ref[...] = jnp.dot(ha, wl_ref[...],
                              preferred_element_type=jnp.float32)
        r2_ref[...] = jnp.dot(ha, wr_ref[...],
                              preferred_element_type=jnp.float32) + b2_ref[...]

    return pl.pallas_call(
        body,
        out_shape=[jax.ShapeDtypeStruct((n, out_d), jnp.float32)] * 2,
    )(agg, c0, c1, r1, gamma, beta, w2_l, b2, w2_r)


def _post(agg, c0, c1, r2):
    def body(agg_ref, c0_ref, c1_ref, r2_ref, out_ref):
        s = agg_ref[0] + agg_ref[1]
        c = c0_ref[...] + c1_ref[...]
        out_ref[...] = s / jnp.maximum(c, 1.0) + r2_ref[...]

    return pl.pallas_call(
        body,
        out_shape=jax.ShapeDtypeStruct(r2.shape, jnp.float32),
    )(agg, c0, c1, r2)


def kernel(x, edge_index, W1_l, b1, W1_r, gamma, beta, W2_l, b2, W2_r):
    n = x.shape[0]
    e = edge_index.shape[1]
    hid = W1_l.shape[1]
    out_d = W2_l.shape[1]

    y1, r1 = _pre(x, W1_l, b1, W1_r)
    agg1, cnt0, cnt1 = _edge_agg(n, e, hid, True)(
        y1, edge_index,
        jnp.zeros((n, hid), jnp.float32), jnp.zeros((n,), jnp.float32))
    c0 = cnt0.reshape(n, 1)
    c1 = cnt1.reshape(n, 1)
    y2, r2 = _mid(agg1, c0, c1, r1, gamma, beta, W2_l, b2, W2_r)
    (agg2,) = _edge_agg(n, e, out_d, False)(
        y2, edge_index, jnp.zeros((n, out_d), jnp.float32))
    return _post(agg2, c0, c1, r2)


# speedup vs baseline: 16.5634x; 1.1102x over previous
import functools

import jax
import jax.numpy as jnp
from jax import lax
from jax.experimental import pallas as pl
from jax.experimental.pallas import tpu as pltpu
from jax.experimental.pallas import tpu_sc as plsc

_NC = 2
_NS = 16
_W = 80
_K = 5


@functools.lru_cache(maxsize=None)
def _edge_agg(n, e, d, with_cnt):
    nw = _NC * _NS
    per_w = e // nw
    gw = _K * _W
    assert e % nw == 0 and per_w % gw == 0
    ngrp = per_w // gw
    assert ngrp % 2 == 1
    per_t = (n // _NS) // 8 * 8
    last_t = n - (_NS - 1) * per_t
    assert per_t % 8 == 0 and last_t > 0

    mesh = plsc.VectorSubcoreMesh(core_axis_name="c", subcore_axis_name="s")

    out_type = [jax.ShapeDtypeStruct((_NC, n, d), jnp.float32)]
    if with_cnt:
        out_type += [jax.ShapeDtypeStruct((n,), jnp.float32)] * 2

    scratch = [
        pltpu.VMEM((_K, 2, _W), jnp.int32),
        pltpu.VMEM((_K, 2, _W), jnp.int32),
        pltpu.VMEM((_K * _W, d), jnp.float32),
        pltpu.VMEM((_K * _W, d), jnp.float32),
        pltpu.SemaphoreType.DMA,
        pltpu.SemaphoreType.DMA,
        pltpu.SemaphoreType.DMA,
        pltpu.SemaphoreType.DMA,
        pltpu.SemaphoreType.DMA,
        pltpu.SemaphoreType.DMA,
        pltpu.VMEM_SHARED((n, d), jnp.float32),
    ]
    if with_cnt:
        scratch.append(pltpu.VMEM_SHARED((n,), jnp.float32))
        scratch.append(pltpu.VMEM((_W,), jnp.float32))

    def body(y_hbm, eidx_hbm, zeros_nd, *rest):
        if with_cnt:
            (zeros_n, agg_out, cnt0_out, cnt1_out,
             ib0, ib1, rb0, rb1, is0, is1, gs0, gs1, ss0, ss1,
             agg_sh, cnt_sh, ones_v) = rest
        else:
            (agg_out, ib0, ib1, rb0, rb1, is0, is1, gs0, gs1, ss0, ss1,
             agg_sh) = rest
        ibuf = (ib0, ib1)
        rbuf = (rb0, rb1)
        isem = (is0, is1)
        gsem = (gs0, gs1)
        ssem = (ss0, ss1)
        cid = lax.axis_index("c")
        sid = lax.axis_index("s")
        ebase = (sid * _NC + cid) * per_w
        row0 = sid * per_t

        @pl.when(sid < _NS - 1)
        def _():
            pltpu.sync_copy(zeros_nd.at[pl.ds(row0, per_t)],
                            agg_sh.at[pl.ds(row0, per_t)])

        @pl.when(sid == _NS - 1)
        def _():
            pltpu.sync_copy(zeros_nd.at[pl.ds((_NS - 1) * per_t, last_t)],
                            agg_sh.at[pl.ds((_NS - 1) * per_t, last_t)])

        if with_cnt:
            @pl.when(sid == 0)
            def _():
                pltpu.sync_copy(zeros_n, cnt_sh)
            for t in range(_W // 16):
                ones_v[pl.ds(t * 16, 16)] = jnp.ones((16,), jnp.float32)
        plsc.subcore_barrier()

        def fire_idx(g, p):
            base = ebase + g * gw
            for k in range(_K):
                pltpu.async_copy(
                    eidx_hbm.at[:, pl.ds(base + k * _W, _W)],
                    ibuf[p].at[k], isem[p])

        def wait_idx(g, p):
            base = ebase + g * gw
            for k in range(_K):
                pltpu.make_async_copy(
                    eidx_hbm.at[:, pl.ds(base + k * _W, _W)],
                    ibuf[p].at[k], isem[p]).wait()

        def fire_gathers(p):
            for k in range(_K):
                pltpu.async_copy(y_hbm.at[ibuf[p].at[k, 0]],
                                 rbuf[p].at[pl.ds(k * _W, _W)], gsem[p])

        def wait_gathers(p):
            for k in range(_K):
                pltpu.make_async_copy(y_hbm.at[ibuf[p].at[k, 0]],
                                      rbuf[p].at[pl.ds(k * _W, _W)],
                                      gsem[p]).wait()

        def fire_scatters(p):
            for k in range(_K):
                pltpu.async_copy(rbuf[p].at[pl.ds(k * _W, _W)],
                                 agg_sh.at[ibuf[p].at[k, 1]], ssem[p],
                                 add=True)
            if with_cnt:
                for k in range(_K):
                    pltpu.async_copy(ones_v, cnt_sh.at[ibuf[p].at[k, 1]],
                                     ssem[p], add=True)

        def drain_scatters(p):
            for k in range(_K):
                pltpu.make_async_copy(rbuf[p].at[pl.ds(k * _W, _W)],
                                      agg_sh.at[ibuf[p].at[k, 1]],
                                      ssem[p]).wait()
            if with_cnt:
                for k in range(_K):
                    pltpu.make_async_copy(ones_v, cnt_sh.at[ibuf[p].at[k, 1]],
                                          ssem[p]).wait()

        def step(g, p, is_last):
            wait_gathers(p)
            fire_scatters(p)
            drain_scatters(p)
            if not is_last:
                wait_idx(g + 1, 1 - p)
                fire_gathers(1 - p)
            if not is_last:
                @pl.when(g + 2 < ngrp)
                def _():
                    fire_idx(g + 2, p)

        fire_idx(0, 0)
        wait_idx(0, 0)
        fire_gathers(0)
        fire_idx(1, 1)

        def loop_body(i, carry):
            step(2 * i, 0, False)
            step(2 * i + 1, 1, False)
            return carry

        lax.fori_loop(0, (ngrp - 1) // 2, loop_body, 0)
        step(ngrp - 1, 0, True)
        plsc.subcore_barrier()

        @pl.when(sid < _NS - 1)
        def _():
            pltpu.sync_copy(agg_sh.at[pl.ds(row0, per_t)],
                            agg_out.at[cid, pl.ds(row0, per_t)])

        @pl.when(sid == _NS - 1)
        def _():
            pltpu.sync_copy(agg_sh.at[pl.ds((_NS - 1) * per_t, last_t)],
                            agg_out.at[cid, pl.ds((_NS - 1) * per_t, last_t)])

        if with_cnt:
            @pl.when((sid == 0) & (cid == 0))
            def _():
                pltpu.sync_copy(cnt_sh, cnt0_out)

            @pl.when((sid == 0) & (cid == 1))
            def _():
                pltpu.sync_copy(cnt_sh, cnt1_out)

    return pl.kernel(
        body, mesh=mesh, out_type=out_type, scratch_types=scratch,
        compiler_params=pltpu.CompilerParams(use_tc_tiling_on_sc=False))


def _pre(x, w_l, b, w_r):
    n = x.shape[0]
    hid = w_l.shape[1]

    def body(x_ref, wl_ref, b_ref, wr_ref, y_ref, r_ref):
        xv = x_ref[...]
        y_ref[...] = jnp.dot(xv, wl_ref[...],
                             preferred_element_type=jnp.float32)
        r_ref[...] = jnp.dot(xv, wr_ref[...],
                             preferred_element_type=jnp.float32) + b_ref[...]

    return pl.pallas_call(
        body,
        out_shape=[jax.ShapeDtypeStruct((n, hid), jnp.float32)] * 2,
    )(x, w_l, b, w_r)


def _mid(agg, c0, c1, r1, gamma, beta, w2_l, b2, w2_r):
    n, hid = r1.shape
    out_d = w2_l.shape[1]

    def body(agg_ref, c0_ref, c1_ref, r1_ref, g_ref, be_ref, wl_ref, b2_ref,
             wr_ref, y2_ref, r2_ref):
        s = agg_ref[0] + agg_ref[1]
        c = c0_ref[...] + c1_ref[...]
        h = s / jnp.maximum(c, 1.0) + r1_ref[...]
        m = jnp.mean(h, axis=0)
        v = jnp.mean((h - m) ** 2, axis=0)
        hn = (h - m) * lax.rsqrt(v + 1e-5) * g_ref[...] + be_ref[...]
        ha = jnp.maximum(hn, 0.0)
        y2_ref[...] = jnp.dot(ha, wl_ref[...],
                              preferred_element_type=jnp.float32)
        r2_ref[...] = jnp.dot(ha, wr_ref[...],
                              preferred_element_type=jnp.float32) + b2_ref[...]

    return pl.pallas_call(
        body,
        out_shape=[jax.ShapeDtypeStruct((n, out_d), jnp.float32)] * 2,
    )(agg, c0, c1, r1, gamma, beta, w2_l, b2, w2_r)


def _post(agg, c0, c1, r2):
    def body(agg_ref, c0_ref, c1_ref, r2_ref, out_ref):
        s = agg_ref[0] + agg_ref[1]
        c = c0_ref[...] + c1_ref[...]
        out_ref[...] = s / jnp.maximum(c, 1.0) + r2_ref[...]

    return pl.pallas_call(
        body,
        out_shape=jax.ShapeDtypeStruct(r2.shape, jnp.float32),
    )(agg, c0, c1, r2)


def kernel(x, edge_index, W1_l, b1, W1_r, gamma, beta, W2_l, b2, W2_r):
    n = x.shape[0]
    e = edge_index.shape[1]
    hid = W1_l.shape[1]
    out_d = W2_l.shape[1]

    y1, r1 = _pre(x, W1_l, b1, W1_r)
    agg1, cnt0, cnt1 = _edge_agg(n, e, hid, True)(
        y1, edge_index,
        jnp.zeros((n, hid), jnp.float32), jnp.zeros((n,), jnp.float32))
    c0 = cnt0.reshape(n, 1)
    c1 = cnt1.reshape(n, 1)
    y2, r2 = _mid(agg1, c0, c1, r1, gamma, beta, W2_l, b2, W2_r)
    (agg2,) = _edge_agg(n, e, out_d, False)(
        y2, edge_index, jnp.zeros((n, out_d), jnp.float32))
    return _post(agg2, c0, c1, r2)
